# baseline (device time: 75645 ns/iter reference)
import functools

import jax
import jax.numpy as jnp
from jax import lax
from jax.experimental import pallas as pl
from jax.experimental.pallas import tpu as pltpu

Z = 4
C = 4
N_DEV = Z * C


def kernel(A, B):
    m, _ = A.shape
    _, n = B.shape
    zch = m // Z
    sch = zch // C

    def body(a_ref, b_ref, out_ref, acc, zrecv, srecv,
             send_sems, zrs_recv, srs_recv, sag_recv, zag_recv):
        my = lax.axis_index("i")
        z = my // C
        c = my % C
        z_right = ((z + 1) % Z) * C + c
        z_left = ((z - 1) % Z) * C + c
        s_right = z * C + (c + 1) % C
        s_left = z * C + (c - 1) % C

        barrier_sem = pltpu.get_barrier_semaphore()
        for nbr in (z_left, z_right, s_left, s_right):
            pl.semaphore_signal(
                barrier_sem, inc=1,
                device_id=(nbr,), device_id_type=pl.DeviceIdType.MESH,
            )
        pl.semaphore_wait(barrier_sem, 4)

        acc[:, :] = jnp.dot(
            a_ref[:, :], b_ref[:, :], preferred_element_type=jnp.float32
        ).astype(jnp.bfloat16)

        for s in range(Z - 1):
            send_p = (z - s) % Z
            recv_p = (z - s - 1) % Z
            rdma = pltpu.make_async_remote_copy(
                src_ref=acc.at[pl.ds(send_p * zch, zch), :],
                dst_ref=zrecv.at[s],
                send_sem=send_sems.at[s % 2],
                recv_sem=zrs_recv.at[s],
                device_id=(z_right,),
                device_id_type=pl.DeviceIdType.MESH,
            )
            rdma.start()
            rdma.wait()
            rows = pl.ds(recv_p * zch, zch)
            acc[rows, :] = acc[rows, :] + zrecv[s]

        zbase = ((z + 1) % Z) * zch

        for s in range(C - 1):
            send_j = (c - s) % C
            recv_j = (c - s - 1) % C
            rdma = pltpu.make_async_remote_copy(
                src_ref=acc.at[pl.ds(zbase + send_j * sch, sch), :],
                dst_ref=srecv.at[s],
                send_sem=send_sems.at[s % 2],
                recv_sem=srs_recv.at[s],
                device_id=(s_right,),
                device_id_type=pl.DeviceIdType.MESH,
            )
            rdma.start()
            rdma.wait()
            rows = pl.ds(zbase + recv_j * sch, sch)
            acc[rows, :] = acc[rows, :] + srecv[s]

        own = pl.ds(zbase + ((c + 1) % C) * sch, sch)
        acc[own, :] = jnp.maximum(acc[own, :], jnp.bfloat16(0))

        for s in range(C - 1):
            send_j = (c + 1 - s) % C
            rows = pl.ds(zbase + send_j * sch, sch)
            rdma = pltpu.make_async_remote_copy(
                src_ref=acc.at[rows, :],
                dst_ref=acc.at[rows, :],
                send_sem=send_sems.at[s % 2],
                recv_sem=sag_recv.at[s],
                device_id=(s_right,),
                device_id_type=pl.DeviceIdType.MESH,
            )
            rdma.start()
            rdma.wait()

        for s in range(Z - 1):
            send_p = (z + 1 - s) % Z
            rows = pl.ds(send_p * zch, zch)
            rdma = pltpu.make_async_remote_copy(
                src_ref=acc.at[rows, :],
                dst_ref=acc.at[rows, :],
                send_sem=send_sems.at[s % 2],
                recv_sem=zag_recv.at[s],
                device_id=(z_right,),
                device_id_type=pl.DeviceIdType.MESH,
            )
            rdma.start()
            rdma.wait()

        out_ref[:, :] = acc[:, :].astype(jnp.float32)

        @functools.partial(
            pl.run_scoped, exit_sem=pltpu.SemaphoreType.REGULAR
        )
        def _(exit_sem):
            for nbr in (z_left, z_right, s_left, s_right):
                pl.semaphore_signal(
                    exit_sem, inc=1,
                    device_id=(nbr,), device_id_type=pl.DeviceIdType.MESH,
                )
            pl.semaphore_wait(exit_sem, 4)

    return pl.pallas_call(
        body,
        out_shape=jax.ShapeDtypeStruct((m, n), jnp.float32),
        in_specs=[
            pl.BlockSpec(memory_space=pltpu.VMEM),
            pl.BlockSpec(memory_space=pltpu.VMEM),
        ],
        out_specs=pl.BlockSpec(memory_space=pltpu.VMEM),
        scratch_shapes=[
            pltpu.VMEM((m, n), jnp.bfloat16),
            pltpu.VMEM((Z - 1, zch, n), jnp.bfloat16),
            pltpu.VMEM((C - 1, sch, n), jnp.bfloat16),
            pltpu.SemaphoreType.DMA((2,)),
            pltpu.SemaphoreType.DMA((Z - 1,)),
            pltpu.SemaphoreType.DMA((C - 1,)),
            pltpu.SemaphoreType.DMA((C - 1,)),
            pltpu.SemaphoreType.DMA((Z - 1,)),
        ],
        compiler_params=pltpu.CompilerParams(collective_id=0),
    )(A, B)


# device time: 67641 ns/iter; 1.1183x vs baseline; 1.1183x over previous
import functools

import jax
import jax.numpy as jnp
from jax import lax
from jax.experimental import pallas as pl
from jax.experimental.pallas import tpu as pltpu

Z = 4
C = 4
N_DEV = Z * C


def kernel(A, B):
    m, _ = A.shape
    _, n = B.shape
    zch = m // Z
    sch = zch // C
    hch = sch // 2

    def body(a_ref, b_ref, out_ref, acc, zrecv, srs_t1, srs_t2,
             send_sems, zrs_r, srs_r, sag_r, zag_r):
        my = lax.axis_index("i")
        z = my // C
        c = my % C
        z_up = ((z + 1) % Z) * C + c
        z_dn = ((z - 1) % Z) * C + c
        s_up = z * C + (c + 1) % C
        s_dn = z * C + (c - 1) % C
        MESH = pl.DeviceIdType.MESH

        barrier_sem = pltpu.get_barrier_semaphore()
        for nbr in (z_dn, z_up, s_dn, s_up):
            pl.semaphore_signal(
                barrier_sem, inc=1, device_id=(nbr,), device_id_type=MESH
            )
        pl.semaphore_wait(barrier_sem, 4)

        def mm(p):
            rows = pl.ds(p * zch, zch)
            acc[rows, :] = jnp.dot(
                a_ref[rows, :], b_ref[:, :],
                preferred_element_type=jnp.float32,
            ).astype(jnp.bfloat16)

        mm(z % Z)
        rdma0 = pltpu.make_async_remote_copy(
            src_ref=acc.at[pl.ds((z % Z) * zch, zch), :],
            dst_ref=zrecv.at[0],
            send_sem=send_sems.at[0],
            recv_sem=zrs_r.at[0],
            device_id=(z_up,),
            device_id_type=MESH,
        )
        rdma0.start()
        mm((z - 1) % Z)
        mm((z - 2) % Z)
        mm((z + 1) % Z)
        rdma0.wait()
        rows = pl.ds(((z - 1) % Z) * zch, zch)
        acc[rows, :] = acc[rows, :] + zrecv[0]

        for s in range(1, Z - 1):
            send_p = (z - s) % Z
            recv_p = (z - s - 1) % Z
            rdma = pltpu.make_async_remote_copy(
                src_ref=acc.at[pl.ds(send_p * zch, zch), :],
                dst_ref=zrecv.at[s],
                send_sem=send_sems.at[s % 2],
                recv_sem=zrs_r.at[s],
                device_id=(z_up,),
                device_id_type=MESH,
            )
            rdma.start()
            rdma.wait()
            rows = pl.ds(recv_p * zch, zch)
            acc[rows, :] = acc[rows, :] + zrecv[s]

        zbase = ((z + 1) % Z) * zch

        def sub(j, half=None):
            j = j % C
            if half is None:
                return pl.ds(zbase + j * sch, sch)
            if half == 0:
                return pl.ds(zbase + j * sch, hch)
            return pl.ds(zbase + j * sch + hch, hch)

        up1 = pltpu.make_async_remote_copy(
            src_ref=acc.at[sub(c + 2, 1), :],
            dst_ref=srs_t1.at[0],
            send_sem=send_sems.at[0],
            recv_sem=srs_r.at[0],
            device_id=(s_up,),
            device_id_type=MESH,
        )
        dn1 = pltpu.make_async_remote_copy(
            src_ref=acc.at[sub(c + 2, 0), :],
            dst_ref=srs_t1.at[1],
            send_sem=send_sems.at[1],
            recv_sem=srs_r.at[1],
            device_id=(s_dn,),
            device_id_type=MESH,
        )
        up1.start()
        dn1.start()
        up1.wait()
        dn1.wait()
        acc[sub(c + 1, 1), :] = acc[sub(c + 1, 1), :] + srs_t1[0]
        acc[sub(c - 1, 0), :] = acc[sub(c - 1, 0), :] + srs_t1[1]

        up2 = pltpu.make_async_remote_copy(
            src_ref=acc.at[sub(c + 1), :],
            dst_ref=srs_t2.at[0],
            send_sem=send_sems.at[0],
            recv_sem=srs_r.at[2],
            device_id=(s_up,),
            device_id_type=MESH,
        )
        dn2 = pltpu.make_async_remote_copy(
            src_ref=acc.at[sub(c - 1), :],
            dst_ref=srs_t2.at[1],
            send_sem=send_sems.at[1],
            recv_sem=srs_r.at[3],
            device_id=(s_dn,),
            device_id_type=MESH,
        )
        up2.start()
        dn2.start()
        up2.wait()
        dn2.wait()
        acc[sub(c), :] = acc[sub(c), :] + srs_t2[0] + srs_t2[1]

        acc[sub(c), :] = jnp.maximum(acc[sub(c), :], jnp.bfloat16(0))

        ag_up1 = pltpu.make_async_remote_copy(
            src_ref=acc.at[sub(c), :],
            dst_ref=acc.at[sub(c), :],
            send_sem=send_sems.at[0],
            recv_sem=sag_r.at[0],
            device_id=(s_up,),
            device_id_type=MESH,
        )
        ag_dn1 = pltpu.make_async_remote_copy(
            src_ref=acc.at[sub(c), :],
            dst_ref=acc.at[sub(c), :],
            send_sem=send_sems.at[1],
            recv_sem=sag_r.at[1],
            device_id=(s_dn,),
            device_id_type=MESH,
        )
        ag_up1.start()
        ag_dn1.start()
        ag_up1.wait()
        ag_dn1.wait()

        ag_up2 = pltpu.make_async_remote_copy(
            src_ref=acc.at[sub(c - 1, 0), :],
            dst_ref=acc.at[sub(c - 1, 0), :],
            send_sem=send_sems.at[0],
            recv_sem=sag_r.at[2],
            device_id=(s_up,),
            device_id_type=MESH,
        )
        ag_dn2 = pltpu.make_async_remote_copy(
            src_ref=acc.at[sub(c + 1, 1), :],
            dst_ref=acc.at[sub(c + 1, 1), :],
            send_sem=send_sems.at[1],
            recv_sem=sag_r.at[3],
            device_id=(s_dn,),
            device_id_type=MESH,
        )
        ag_up2.start()
        ag_dn2.start()
        ag_up2.wait()
        ag_dn2.wait()

        for s in range(Z - 1):
            send_p = (z + 1 - s) % Z
            rows = pl.ds(send_p * zch, zch)
            rdma = pltpu.make_async_remote_copy(
                src_ref=acc.at[rows, :],
                dst_ref=acc.at[rows, :],
                send_sem=send_sems.at[s % 2],
                recv_sem=zag_r.at[s],
                device_id=(z_up,),
                device_id_type=MESH,
            )
            rdma.start()
            rdma.wait()

        out_ref[:, :] = acc[:, :].astype(jnp.float32)

        @functools.partial(
            pl.run_scoped, exit_sem=pltpu.SemaphoreType.REGULAR
        )
        def _(exit_sem):
            for nbr in (z_dn, z_up, s_dn, s_up):
                pl.semaphore_signal(
                    exit_sem, inc=1, device_id=(nbr,), device_id_type=MESH
                )
            pl.semaphore_wait(exit_sem, 4)

    return pl.pallas_call(
        body,
        out_shape=jax.ShapeDtypeStruct((m, n), jnp.float32),
        in_specs=[
            pl.BlockSpec(memory_space=pltpu.VMEM),
            pl.BlockSpec(memory_space=pltpu.VMEM),
        ],
        out_specs=pl.BlockSpec(memory_space=pltpu.VMEM),
        scratch_shapes=[
            pltpu.VMEM((m, n), jnp.bfloat16),
            pltpu.VMEM((Z - 1, zch, n), jnp.bfloat16),
            pltpu.VMEM((2, hch, n), jnp.bfloat16),
            pltpu.VMEM((2, sch, n), jnp.bfloat16),
            pltpu.SemaphoreType.DMA((2,)),
            pltpu.SemaphoreType.DMA((Z - 1,)),
            pltpu.SemaphoreType.DMA((4,)),
            pltpu.SemaphoreType.DMA((4,)),
            pltpu.SemaphoreType.DMA((Z - 1,)),
        ],
        compiler_params=pltpu.CompilerParams(collective_id=0),
    )(A, B)


# device time: 54996 ns/iter; 1.3755x vs baseline; 1.2299x over previous
import functools

import jax
import jax.numpy as jnp
from jax import lax
from jax.experimental import pallas as pl
from jax.experimental.pallas import tpu as pltpu

Z = 4
C = 4
N_DEV = Z * C


def kernel(A, B):
    m, _ = A.shape
    _, n = B.shape
    qch = m // C
    hq = qch // 2
    pch = qch // Z

    def body(a_ref, b_ref, out_ref, acc, srs_t1, srs_t2, zrecv,
             send_sems, srs_r, zrs_r, zag_r, sag_r):
        my = lax.axis_index("i")
        z = my // C
        c = my % C
        z_up = ((z + 1) % Z) * C + c
        z_dn = ((z - 1) % Z) * C + c
        s_up = z * C + (c + 1) % C
        s_dn = z * C + (c - 1) % C
        MESH = pl.DeviceIdType.MESH

        barrier_sem = pltpu.get_barrier_semaphore()
        for nbr in (z_dn, z_up, s_dn, s_up):
            pl.semaphore_signal(
                barrier_sem, inc=1, device_id=(nbr,), device_id_type=MESH
            )
        pl.semaphore_wait(barrier_sem, 4)

        def quarter(q, half=None):
            q = q % C
            if half is None:
                return pl.ds(q * qch, qch)
            if half == 0:
                return pl.ds(q * qch, hq)
            return pl.ds(q * qch + hq, hq)

        def zrows(j):
            return pl.ds(c * qch + (j % Z) * pch, pch)

        def mm(q):
            rows = quarter(q)
            acc[rows, :] = jnp.dot(
                a_ref[rows, :], b_ref[:, :],
                preferred_element_type=jnp.float32,
            ).astype(jnp.bfloat16)

        mm(c + 2)
        up1 = pltpu.make_async_remote_copy(
            src_ref=acc.at[quarter(c + 2, 1), :],
            dst_ref=srs_t1.at[0],
            send_sem=send_sems.at[0],
            recv_sem=srs_r.at[0],
            device_id=(s_up,),
            device_id_type=MESH,
        )
        dn1 = pltpu.make_async_remote_copy(
            src_ref=acc.at[quarter(c + 2, 0), :],
            dst_ref=srs_t1.at[1],
            send_sem=send_sems.at[1],
            recv_sem=srs_r.at[1],
            device_id=(s_dn,),
            device_id_type=MESH,
        )
        up1.start()
        dn1.start()
        mm(c + 1)
        mm(c - 1)
        up1.wait()
        dn1.wait()
        acc[quarter(c + 1, 1), :] = acc[quarter(c + 1, 1), :] + srs_t1[0]
        acc[quarter(c - 1, 0), :] = acc[quarter(c - 1, 0), :] + srs_t1[1]

        up2 = pltpu.make_async_remote_copy(
            src_ref=acc.at[quarter(c + 1), :],
            dst_ref=srs_t2.at[0],
            send_sem=send_sems.at[0],
            recv_sem=srs_r.at[2],
            device_id=(s_up,),
            device_id_type=MESH,
        )
        dn2 = pltpu.make_async_remote_copy(
            src_ref=acc.at[quarter(c - 1), :],
            dst_ref=srs_t2.at[1],
            send_sem=send_sems.at[1],
            recv_sem=srs_r.at[3],
            device_id=(s_dn,),
            device_id_type=MESH,
        )
        up2.start()
        dn2.start()
        mm(c)
        up2.wait()
        dn2.wait()
        acc[quarter(c), :] = acc[quarter(c), :] + srs_t2[0] + srs_t2[1]

        for s in range(Z - 1):
            rdma = pltpu.make_async_remote_copy(
                src_ref=acc.at[zrows(z - s), :],
                dst_ref=zrecv.at[s],
                send_sem=send_sems.at[s % 2],
                recv_sem=zrs_r.at[s],
                device_id=(z_up,),
                device_id_type=MESH,
            )
            rdma.start()
            rdma.wait()
            rows = zrows(z - s - 1)
            acc[rows, :] = acc[rows, :] + zrecv[s]

        acc[zrows(z + 1), :] = jnp.maximum(
            acc[zrows(z + 1), :], jnp.bfloat16(0)
        )

        for s in range(Z - 1):
            rows = zrows(z + 1 - s)
            rdma = pltpu.make_async_remote_copy(
                src_ref=acc.at[rows, :],
                dst_ref=acc.at[rows, :],
                send_sem=send_sems.at[s % 2],
                recv_sem=zag_r.at[s],
                device_id=(z_up,),
                device_id_type=MESH,
            )
            rdma.start()
            rdma.wait()

        ag_up1 = pltpu.make_async_remote_copy(
            src_ref=acc.at[quarter(c), :],
            dst_ref=acc.at[quarter(c), :],
            send_sem=send_sems.at[0],
            recv_sem=sag_r.at[0],
            device_id=(s_up,),
            device_id_type=MESH,
        )
        ag_dn1 = pltpu.make_async_remote_copy(
            src_ref=acc.at[quarter(c), :],
            dst_ref=acc.at[quarter(c), :],
            send_sem=send_sems.at[1],
            recv_sem=sag_r.at[1],
            device_id=(s_dn,),
            device_id_type=MESH,
        )
        ag_up1.start()
        ag_dn1.start()
        ag_up1.wait()
        ag_dn1.wait()

        ag_up2 = pltpu.make_async_remote_copy(
            src_ref=acc.at[quarter(c - 1, 0), :],
            dst_ref=acc.at[quarter(c - 1, 0), :],
            send_sem=send_sems.at[0],
            recv_sem=sag_r.at[2],
            device_id=(s_up,),
            device_id_type=MESH,
        )
        ag_dn2 = pltpu.make_async_remote_copy(
            src_ref=acc.at[quarter(c + 1, 1), :],
            dst_ref=acc.at[quarter(c + 1, 1), :],
            send_sem=send_sems.at[1],
            recv_sem=sag_r.at[3],
            device_id=(s_dn,),
            device_id_type=MESH,
        )
        ag_up2.start()
        ag_dn2.start()
        ag_up2.wait()
        ag_dn2.wait()

        out_ref[:, :] = acc[:, :].astype(jnp.float32)

        @functools.partial(
            pl.run_scoped, exit_sem=pltpu.SemaphoreType.REGULAR
        )
        def _(exit_sem):
            for nbr in (z_dn, z_up, s_dn, s_up):
                pl.semaphore_signal(
                    exit_sem, inc=1, device_id=(nbr,), device_id_type=MESH
                )
            pl.semaphore_wait(exit_sem, 4)

    return pl.pallas_call(
        body,
        out_shape=jax.ShapeDtypeStruct((m, n), jnp.float32),
        in_specs=[
            pl.BlockSpec(memory_space=pltpu.VMEM),
            pl.BlockSpec(memory_space=pltpu.VMEM),
        ],
        out_specs=pl.BlockSpec(memory_space=pltpu.VMEM),
        scratch_shapes=[
            pltpu.VMEM((m, n), jnp.bfloat16),
            pltpu.VMEM((2, hq, n), jnp.bfloat16),
            pltpu.VMEM((2, qch, n), jnp.bfloat16),
            pltpu.VMEM((Z - 1, pch, n), jnp.bfloat16),
            pltpu.SemaphoreType.DMA((2,)),
            pltpu.SemaphoreType.DMA((4,)),
            pltpu.SemaphoreType.DMA((Z - 1,)),
            pltpu.SemaphoreType.DMA((Z - 1,)),
            pltpu.SemaphoreType.DMA((4,)),
        ],
        compiler_params=pltpu.CompilerParams(collective_id=0),
    )(A, B)


# device time: 50966 ns/iter; 1.4842x vs baseline; 1.0791x over previous
import functools

import jax
import jax.numpy as jnp
from jax import lax
from jax.experimental import pallas as pl
from jax.experimental.pallas import tpu as pltpu

Z = 4
C = 4
N_DEV = Z * C


def kernel(A, B):
    m, _ = A.shape
    _, n = B.shape
    qch = m // C
    hq = qch // 2
    pch = qch // Z

    def body(a_ref, b_ref, out_ref, acc, srs_t1, srs_t2, zrecv,
             send_sems, srs_r, zrs_r, zag_r, sag_r):
        my = lax.axis_index("i")
        z = my // C
        c = my % C
        s_up = z * C + (c + 1) % C
        s_dn = z * C + (c - 1) % C
        MESH = pl.DeviceIdType.MESH

        col = [((z + k) % Z) * C + c for k in range(1, Z)]
        barrier_sem = pltpu.get_barrier_semaphore()
        for nbr in [s_dn, s_up] + col:
            pl.semaphore_signal(
                barrier_sem, inc=1, device_id=(nbr,), device_id_type=MESH
            )
        pl.semaphore_wait(barrier_sem, 5)

        def quarter(q, half=None):
            q = q % C
            if half is None:
                return pl.ds(q * qch, qch)
            if half == 0:
                return pl.ds(q * qch, hq)
            return pl.ds(q * qch + hq, hq)

        def zrows(j):
            return pl.ds(c * qch + (j % Z) * pch, pch)

        def mm(rows):
            acc[rows, :] = jnp.dot(
                a_ref[rows, :], b_ref[:, :],
                preferred_element_type=jnp.float32,
            ).astype(jnp.bfloat16)

        mm(quarter(c + 2, 1))
        up1 = pltpu.make_async_remote_copy(
            src_ref=acc.at[quarter(c + 2, 1), :],
            dst_ref=srs_t1.at[0],
            send_sem=send_sems.at[0],
            recv_sem=srs_r.at[0],
            device_id=(s_up,),
            device_id_type=MESH,
        )
        up1.start()
        mm(quarter(c + 2, 0))
        dn1 = pltpu.make_async_remote_copy(
            src_ref=acc.at[quarter(c + 2, 0), :],
            dst_ref=srs_t1.at[1],
            send_sem=send_sems.at[1],
            recv_sem=srs_r.at[1],
            device_id=(s_dn,),
            device_id_type=MESH,
        )
        dn1.start()
        mm(quarter(c + 1))
        mm(quarter(c - 1))
        up1.wait()
        dn1.wait()
        acc[quarter(c + 1, 1), :] = acc[quarter(c + 1, 1), :] + srs_t1[0]
        acc[quarter(c - 1, 0), :] = acc[quarter(c - 1, 0), :] + srs_t1[1]

        up2 = pltpu.make_async_remote_copy(
            src_ref=acc.at[quarter(c + 1), :],
            dst_ref=srs_t2.at[0],
            send_sem=send_sems.at[0],
            recv_sem=srs_r.at[2],
            device_id=(s_up,),
            device_id_type=MESH,
        )
        dn2 = pltpu.make_async_remote_copy(
            src_ref=acc.at[quarter(c - 1), :],
            dst_ref=srs_t2.at[1],
            send_sem=send_sems.at[1],
            recv_sem=srs_r.at[3],
            device_id=(s_dn,),
            device_id_type=MESH,
        )
        up2.start()
        dn2.start()
        mm(quarter(c))
        up2.wait()
        dn2.wait()
        acc[quarter(c), :] = acc[quarter(c), :] + srs_t2[0] + srs_t2[1]

        zrs = []
        for k in range(1, Z):
            rdma = pltpu.make_async_remote_copy(
                src_ref=acc.at[zrows(z + k), :],
                dst_ref=zrecv.at[k - 1],
                send_sem=send_sems.at[k - 1],
                recv_sem=zrs_r.at[k - 1],
                device_id=(col[k - 1],),
                device_id_type=MESH,
            )
            rdma.start()
            zrs.append(rdma)
        for rdma in zrs:
            rdma.wait()

        acc[zrows(z), :] = jnp.maximum(
            acc[zrows(z), :] + zrecv[0] + zrecv[1] + zrecv[2],
            jnp.bfloat16(0),
        )

        zag = []
        for k in range(1, Z):
            rdma = pltpu.make_async_remote_copy(
                src_ref=acc.at[zrows(z), :],
                dst_ref=acc.at[zrows(z), :],
                send_sem=send_sems.at[k - 1],
                recv_sem=zag_r.at[k - 1],
                device_id=(col[k - 1],),
                device_id_type=MESH,
            )
            rdma.start()
            zag.append(rdma)
        for rdma in zag:
            rdma.wait()

        ag_up1 = pltpu.make_async_remote_copy(
            src_ref=acc.at[quarter(c), :],
            dst_ref=acc.at[quarter(c), :],
            send_sem=send_sems.at[0],
            recv_sem=sag_r.at[0],
            device_id=(s_up,),
            device_id_type=MESH,
        )
        ag_dn1 = pltpu.make_async_remote_copy(
            src_ref=acc.at[quarter(c), :],
            dst_ref=acc.at[quarter(c), :],
            send_sem=send_sems.at[1],
            recv_sem=sag_r.at[1],
            device_id=(s_dn,),
            device_id_type=MESH,
        )
        ag_up1.start()
        ag_dn1.start()
        rows = quarter(c)
        out_ref[rows, :] = acc[rows, :].astype(jnp.float32)
        ag_up1.wait()
        ag_dn1.wait()

        ag_up2 = pltpu.make_async_remote_copy(
            src_ref=acc.at[quarter(c - 1, 0), :],
            dst_ref=acc.at[quarter(c - 1, 0), :],
            send_sem=send_sems.at[0],
            recv_sem=sag_r.at[2],
            device_id=(s_up,),
            device_id_type=MESH,
        )
        ag_dn2 = pltpu.make_async_remote_copy(
            src_ref=acc.at[quarter(c + 1, 1), :],
            dst_ref=acc.at[quarter(c + 1, 1), :],
            send_sem=send_sems.at[1],
            recv_sem=sag_r.at[3],
            device_id=(s_dn,),
            device_id_type=MESH,
        )
        ag_up2.start()
        ag_dn2.start()
        for q in (c + 1, c - 1):
            rows = quarter(q)
            out_ref[rows, :] = acc[rows, :].astype(jnp.float32)
        ag_up2.wait()
        ag_dn2.wait()
        rows = quarter(c + 2)
        out_ref[rows, :] = acc[rows, :].astype(jnp.float32)

        @functools.partial(
            pl.run_scoped, exit_sem=pltpu.SemaphoreType.REGULAR
        )
        def _(exit_sem):
            for nbr in [s_dn, s_up] + col:
                pl.semaphore_signal(
                    exit_sem, inc=1, device_id=(nbr,), device_id_type=MESH
                )
            pl.semaphore_wait(exit_sem, 5)

    return pl.pallas_call(
        body,
        out_shape=jax.ShapeDtypeStruct((m, n), jnp.float32),
        in_specs=[
            pl.BlockSpec(memory_space=pltpu.VMEM),
            pl.BlockSpec(memory_space=pltpu.VMEM),
        ],
        out_specs=pl.BlockSpec(memory_space=pltpu.VMEM),
        scratch_shapes=[
            pltpu.VMEM((m, n), jnp.bfloat16),
            pltpu.VMEM((2, hq, n), jnp.bfloat16),
            pltpu.VMEM((2, qch, n), jnp.bfloat16),
            pltpu.VMEM((Z - 1, pch, n), jnp.bfloat16),
            pltpu.SemaphoreType.DMA((3,)),
            pltpu.SemaphoreType.DMA((4,)),
            pltpu.SemaphoreType.DMA((Z - 1,)),
            pltpu.SemaphoreType.DMA((Z - 1,)),
            pltpu.SemaphoreType.DMA((4,)),
        ],
        compiler_params=pltpu.CompilerParams(collective_id=0),
    )(A, B)


# device time: 41082 ns/iter; 1.8413x vs baseline; 1.2406x over previous
import jax
import jax.numpy as jnp
from jax import lax
from jax.experimental import pallas as pl
from jax.experimental.pallas import tpu as pltpu

Z = 4
C = 4
N_DEV = Z * C


def kernel(A, B):
    m, _ = A.shape
    _, n = B.shape
    qch = m // C
    hq = qch // 2
    pch = qch // Z

    def body(a_ref, b_ref, out_ref, acc, srs_t1, srs_t2, zrecv,
             send_sems, srs1_r, srs2_r, zrs_r, zag_r, sag1_r, sag2_r):
        my = lax.axis_index("i")
        z = my // C
        c = my % C
        s_up = z * C + (c + 1) % C
        s_dn = z * C + (c - 1) % C
        MESH = pl.DeviceIdType.MESH

        col = [((z + k) % Z) * C + c for k in range(1, Z)]
        barrier_sem = pltpu.get_barrier_semaphore()
        for nbr in [s_dn, s_up] + col:
            pl.semaphore_signal(
                barrier_sem, inc=1, device_id=(nbr,), device_id_type=MESH
            )
        pl.semaphore_wait(barrier_sem, 5)

        def quarter(q, half=None):
            q = q % C
            if half is None:
                return pl.ds(q * qch, qch)
            return pl.ds(q * qch + half * hq, hq)

        def piece(q, j):
            return pl.ds((q % C) * qch + (j % Z) * pch, pch)

        def poff(j):
            return pl.ds((j % Z) * pch, pch)

        def mm(rows):
            acc[rows, :] = jnp.dot(
                a_ref[rows, :], b_ref[:, :],
                preferred_element_type=jnp.float32,
            ).astype(jnp.bfloat16)

        mm(quarter(c + 2, 1))
        up1 = pltpu.make_async_remote_copy(
            src_ref=acc.at[quarter(c + 2, 1), :],
            dst_ref=srs_t1.at[0],
            send_sem=send_sems.at[0],
            recv_sem=srs1_r.at[0],
            device_id=(s_up,),
            device_id_type=MESH,
        )
        up1.start()
        mm(quarter(c + 2, 0))
        dn1 = pltpu.make_async_remote_copy(
            src_ref=acc.at[quarter(c + 2, 0), :],
            dst_ref=srs_t1.at[1],
            send_sem=send_sems.at[1],
            recv_sem=srs1_r.at[1],
            device_id=(s_dn,),
            device_id_type=MESH,
        )
        dn1.start()
        mm(quarter(c + 1))
        mm(quarter(c - 1))
        up1.wait()
        dn1.wait()
        acc[quarter(c + 1, 1), :] = acc[quarter(c + 1, 1), :] + srs_t1[0]
        acc[quarter(c - 1, 0), :] = acc[quarter(c - 1, 0), :] + srs_t1[1]

        t2 = []
        for i in range(Z):
            up2 = pltpu.make_async_remote_copy(
                src_ref=acc.at[piece(c + 1, z + i), :],
                dst_ref=srs_t2.at[0, poff(z + i)],
                send_sem=send_sems.at[2 + i],
                recv_sem=srs2_r.at[i],
                device_id=(s_up,),
                device_id_type=MESH,
            )
            dn2 = pltpu.make_async_remote_copy(
                src_ref=acc.at[piece(c - 1, z + i), :],
                dst_ref=srs_t2.at[1, poff(z + i)],
                send_sem=send_sems.at[6 + i],
                recv_sem=srs2_r.at[4 + i],
                device_id=(s_dn,),
                device_id_type=MESH,
            )
            up2.start()
            dn2.start()
            t2.append((up2, dn2))
        mm(quarter(c))

        zsends = []
        for i in range(Z):
            up2, dn2 = t2[i]
            up2.wait()
            dn2.wait()
            rows = piece(c, z + i)
            acc[rows, :] = (
                acc[rows, :] + srs_t2[0, poff(z + i)] + srs_t2[1, poff(z + i)]
            )
            if i > 0:
                rdma = pltpu.make_async_remote_copy(
                    src_ref=acc.at[rows, :],
                    dst_ref=zrecv.at[i - 1],
                    send_sem=send_sems.at[10 + i],
                    recv_sem=zrs_r.at[i - 1],
                    device_id=(col[i - 1],),
                    device_id_type=MESH,
                )
                rdma.start()
                zsends.append(rdma)

        for rdma in zsends:
            rdma.wait()
        acc[piece(c, z), :] = jnp.maximum(
            acc[piece(c, z), :] + zrecv[0] + zrecv[1] + zrecv[2],
            jnp.bfloat16(0),
        )

        zag = []
        for k in range(1, Z):
            rdma = pltpu.make_async_remote_copy(
                src_ref=acc.at[piece(c, z), :],
                dst_ref=acc.at[piece(c, z), :],
                send_sem=send_sems.at[1 + k],
                recv_sem=zag_r.at[k - 1],
                device_id=(col[k - 1],),
                device_id_type=MESH,
            )
            rdma.start()
            zag.append(rdma)

        sag1 = []
        for i in range(Z):
            if i > 0:
                zag[i - 1].wait()
            rows = piece(c, z - i)
            ag_up = pltpu.make_async_remote_copy(
                src_ref=acc.at[rows, :],
                dst_ref=acc.at[rows, :],
                send_sem=send_sems.at[5 + i],
                recv_sem=sag1_r.at[i],
                device_id=(s_up,),
                device_id_type=MESH,
            )
            ag_dn = pltpu.make_async_remote_copy(
                src_ref=acc.at[rows, :],
                dst_ref=acc.at[rows, :],
                send_sem=send_sems.at[9 + i],
                recv_sem=sag1_r.at[4 + i],
                device_id=(s_dn,),
                device_id_type=MESH,
            )
            ag_up.start()
            ag_dn.start()
            sag1.append((ag_up, ag_dn))

        rows = quarter(c)
        out_ref[rows, :] = acc[rows, :].astype(jnp.float32)

        for ag_up, ag_dn in sag1:
            ag_up.wait()
            ag_dn.wait()

        ag_up2 = pltpu.make_async_remote_copy(
            src_ref=acc.at[quarter(c - 1, 0), :],
            dst_ref=acc.at[quarter(c - 1, 0), :],
            send_sem=send_sems.at[0],
            recv_sem=sag2_r.at[0],
            device_id=(s_up,),
            device_id_type=MESH,
        )
        ag_dn2 = pltpu.make_async_remote_copy(
            src_ref=acc.at[quarter(c + 1, 1), :],
            dst_ref=acc.at[quarter(c + 1, 1), :],
            send_sem=send_sems.at[1],
            recv_sem=sag2_r.at[1],
            device_id=(s_dn,),
            device_id_type=MESH,
        )
        ag_up2.start()
        ag_dn2.start()
        for q in (c + 1, c - 1):
            rows = quarter(q)
            out_ref[rows, :] = acc[rows, :].astype(jnp.float32)
        ag_up2.wait()
        ag_dn2.wait()
        rows = quarter(c + 2)
        out_ref[rows, :] = acc[rows, :].astype(jnp.float32)

    return pl.pallas_call(
        body,
        out_shape=jax.ShapeDtypeStruct((m, n), jnp.float32),
        in_specs=[
            pl.BlockSpec(memory_space=pltpu.VMEM),
            pl.BlockSpec(memory_space=pltpu.VMEM),
        ],
        out_specs=pl.BlockSpec(memory_space=pltpu.VMEM),
        scratch_shapes=[
            pltpu.VMEM((m, n), jnp.bfloat16),
            pltpu.VMEM((2, hq, n), jnp.bfloat16),
            pltpu.VMEM((2, qch, n), jnp.bfloat16),
            pltpu.VMEM((Z - 1, pch, n), jnp.bfloat16),
            pltpu.SemaphoreType.DMA((14,)),
            pltpu.SemaphoreType.DMA((2,)),
            pltpu.SemaphoreType.DMA((8,)),
            pltpu.SemaphoreType.DMA((Z - 1,)),
            pltpu.SemaphoreType.DMA((Z - 1,)),
            pltpu.SemaphoreType.DMA((8,)),
            pltpu.SemaphoreType.DMA((2,)),
        ],
        compiler_params=pltpu.CompilerParams(collective_id=0),
    )(A, B)
